# Initial kernel scaffold; baseline (speedup 1.0000x reference)
import jax
import jax.numpy as jnp
from jax.experimental import pallas as pl


def kernel(x, table, W, b):
    raise NotImplementedError("write your pallas kernel here")


# speedup vs baseline: 1.0633x; 1.0633x over previous
import functools

import jax
import jax.numpy as jnp
from jax import lax
from jax.experimental import pallas as pl
from jax.experimental.pallas import tpu as pltpu
from jax.experimental.pallas import tpu_sc as plsc

B = 16384
L = 50
D = 64
N_ROWS = B * L
NC, NS = 2, 16
NW = NC * NS
ROWS_PER_W = N_ROWS // NW
CHUNK = 128
N_CHUNKS = ROWS_PER_W // CHUNK

_sc_mesh = plsc.VectorSubcoreMesh(
    core_axis_name="c", subcore_axis_name="s", num_cores=NC, num_subcores=NS
)


@functools.partial(
    pl.kernel,
    out_type=jax.ShapeDtypeStruct((N_ROWS, D), jnp.float32),
    mesh=_sc_mesh,
    scratch_types=[
        pltpu.VMEM((N_CHUNKS, CHUNK), jnp.int32),
        pltpu.VMEM((CHUNK, D), jnp.float32),
        pltpu.SemaphoreType.DMA,
    ],
    compiler_params=pltpu.CompilerParams(use_tc_tiling_on_sc=False),
)
def _sc_gather(table_hbm, idx_hbm, out_hbm, idx_v, rows_v, sem):
    wid = lax.axis_index("s") * NC + lax.axis_index("c")
    base = wid * ROWS_PER_W
    pltpu.sync_copy(idx_hbm.at[wid], idx_v)

    def body(j, carry):
        pltpu.async_copy(table_hbm.at[idx_v.at[j]], rows_v, sem).wait()
        pltpu.sync_copy(rows_v, out_hbm.at[pl.ds(base + j * CHUNK, CHUNK)])
        return carry

    lax.fori_loop(0, N_CHUNKS, body, 0)


BLK = 4096


def _proj_body(emb_ref, wt_ref, b_ref, out_ref):
    out_ref[...] = (
        jnp.dot(emb_ref[...], wt_ref[...], preferred_element_type=jnp.float32)
        + b_ref[...]
    )


def _project(emb, wt, b2):
    return pl.pallas_call(
        _proj_body,
        grid=(N_ROWS // BLK,),
        in_specs=[
            pl.BlockSpec((BLK, D), lambda i: (i, 0)),
            pl.BlockSpec((D, D), lambda i: (0, 0)),
            pl.BlockSpec((1, D), lambda i: (0, 0)),
        ],
        out_specs=pl.BlockSpec((BLK, D), lambda i: (i, 0)),
        out_shape=jax.ShapeDtypeStruct((N_ROWS, D), jnp.float32),
    )(emb, wt, b2)


def kernel(x, table, W, b):
    idx3 = x.reshape(NW, N_CHUNKS, CHUNK)
    emb = _sc_gather(table, idx3)
    out = _project(emb, W.T, b.reshape(1, D))
    return out.reshape(B, L, D)


# speedup vs baseline: 1.2685x; 1.1929x over previous
import functools

import jax
import jax.numpy as jnp
from jax import lax
from jax.experimental import pallas as pl
from jax.experimental.pallas import tpu as pltpu
from jax.experimental.pallas import tpu_sc as plsc

B = 16384
L = 50
D = 64
N_ROWS = B * L
NC, NS = 2, 16
NW = NC * NS
ROWS_PER_W = N_ROWS // NW
CHUNK = 128
N_CHUNKS = ROWS_PER_W // CHUNK

K = 4
NG = N_CHUNKS // K

_sc_mesh = plsc.VectorSubcoreMesh(
    core_axis_name="c", subcore_axis_name="s", num_cores=NC, num_subcores=NS
)


@functools.partial(
    pl.kernel,
    out_type=jax.ShapeDtypeStruct((N_ROWS, D), jnp.float32),
    mesh=_sc_mesh,
    scratch_types=[
        pltpu.VMEM((N_CHUNKS, CHUNK), jnp.int32),
        [pltpu.VMEM((CHUNK, D), jnp.float32)] * K,
        [pltpu.VMEM((CHUNK, D), jnp.float32)] * K,
        pltpu.SemaphoreType.DMA,
        pltpu.SemaphoreType.DMA,
        pltpu.SemaphoreType.DMA,
        pltpu.SemaphoreType.DMA,
    ],
    compiler_params=pltpu.CompilerParams(use_tc_tiling_on_sc=False),
)
def _sc_gather(table_hbm, idx_hbm, out_hbm, idx_v, bank0, bank1, sg0, sg1, sc0, sc1):
    wid = lax.axis_index("s") * NC + lax.axis_index("c")
    base = wid * ROWS_PER_W
    banks = (bank0, bank1)
    sg = (sg0, sg1)
    sc = (sc0, sc1)
    pltpu.sync_copy(idx_hbm.at[wid], idx_v)

    def fire_gathers(g, bk):
        for i in range(K):
            pltpu.async_copy(table_hbm.at[idx_v.at[g * K + i]], banks[bk][i], sg[bk])

    def drain(bk, sem_bank):
        for i in range(K):
            pltpu.make_async_copy(
                out_hbm.at[pl.ds(0, CHUNK)], banks[bk][i], sem_bank[bk]
            ).wait()

    def fire_copyouts(g, bk):
        for i in range(K):
            pltpu.async_copy(
                banks[bk][i], out_hbm.at[pl.ds(base + (g * K + i) * CHUNK, CHUNK)],
                sc[bk],
            )

    fire_gathers(0, 0)

    def body(g, carry):
        drain(0, sg)

        @pl.when(g > 0)
        def _():
            drain(1, sc)

        fire_gathers(g + 1, 1)
        fire_copyouts(g, 0)
        drain(1, sg)
        drain(0, sc)

        @pl.when(g + 2 < NG)
        def _():
            fire_gathers(g + 2, 0)

        fire_copyouts(g + 1, 1)
        return carry

    lax.fori_loop(0, NG // 2, lambda t, c: body(t * 2, c), 0)
    drain(1, sc)


SB = 64
RB = SB * L
N_BLK = B // SB


def _proj_body(emb_hbm, wt_ref, b_ref, out_ref, ebuf, sems):
    i = pl.program_id(0)

    def copy_in(j, slot):
        return pltpu.make_async_copy(
            emb_hbm.at[pl.ds(j * RB, RB), :], ebuf.at[slot], sems.at[slot]
        )

    @pl.when(i == 0)
    def _():
        copy_in(0, 0).start()

    @pl.when(i + 1 < N_BLK)
    def _():
        copy_in(i + 1, (i + 1) % 2).start()

    copy_in(i, i % 2).wait()
    e = ebuf[i % 2]
    p = jnp.dot(e, wt_ref[...], preferred_element_type=jnp.float32) + b_ref[...]
    out_ref[...] = p.reshape(SB, L, D)


def _project(emb, wt, b2):
    return pl.pallas_call(
        _proj_body,
        grid=(N_BLK,),
        in_specs=[
            pl.BlockSpec(memory_space=pl.ANY),
            pl.BlockSpec((D, D), lambda i: (0, 0)),
            pl.BlockSpec((1, D), lambda i: (0, 0)),
        ],
        out_specs=pl.BlockSpec((SB, L, D), lambda i: (i, 0, 0)),
        out_shape=jax.ShapeDtypeStruct((B, L, D), jnp.float32),
        scratch_shapes=[
            pltpu.VMEM((2, RB, D), jnp.float32),
            pltpu.SemaphoreType.DMA((2,)),
        ],
    )(emb, wt, b2)


def kernel(x, table, W, b):
    idx3 = x.reshape(NW, N_CHUNKS, CHUNK)
    emb = _sc_gather(table, idx3)
    return _project(emb, W.T, b.reshape(1, D))


# speedup vs baseline: 1.5096x; 1.1901x over previous
import functools

import jax
import jax.numpy as jnp
from jax import lax
from jax.experimental import pallas as pl
from jax.experimental.pallas import tpu as pltpu
from jax.experimental.pallas import tpu_sc as plsc

B = 16384
L = 50
D = 64
N_ROWS = B * L
NC, NS = 2, 16
NW = NC * NS
ROWS_PER_W = N_ROWS // NW
CHUNK = 128
N_CHUNKS = ROWS_PER_W // CHUNK

K = 4
NG = N_CHUNKS // K

_sc_mesh = plsc.VectorSubcoreMesh(
    core_axis_name="c", subcore_axis_name="s", num_cores=NC, num_subcores=NS
)


@functools.partial(
    pl.kernel,
    out_type=jax.ShapeDtypeStruct((N_ROWS, D), jnp.float32),
    mesh=_sc_mesh,
    scratch_types=[
        pltpu.VMEM((N_CHUNKS, CHUNK), jnp.int32),
        [pltpu.VMEM((CHUNK, D), jnp.float32)] * K,
        [pltpu.VMEM((CHUNK, D), jnp.float32)] * K,
        pltpu.SemaphoreType.DMA,
        pltpu.SemaphoreType.DMA,
        pltpu.SemaphoreType.DMA,
        pltpu.SemaphoreType.DMA,
    ],
    compiler_params=pltpu.CompilerParams(use_tc_tiling_on_sc=False),
)
def _sc_gather(table_hbm, idx_hbm, out_hbm, idx_v, bank0, bank1, sg0, sg1, sc0, sc1):
    wid = lax.axis_index("s") * NC + lax.axis_index("c")
    base = wid * ROWS_PER_W
    banks = (bank0, bank1)
    sg = (sg0, sg1)
    sc = (sc0, sc1)
    pltpu.sync_copy(idx_hbm.at[wid], idx_v)

    def fire_gathers(g, bk):
        for i in range(K):
            pltpu.async_copy(table_hbm.at[idx_v.at[g * K + i]], banks[bk][i], sg[bk])

    def drain(bk, sem_bank):
        for i in range(K):
            pltpu.make_async_copy(
                out_hbm.at[pl.ds(0, CHUNK)], banks[bk][i], sem_bank[bk]
            ).wait()

    def fire_copyouts(g, bk):
        for i in range(K):
            pltpu.async_copy(
                banks[bk][i], out_hbm.at[pl.ds(base + (g * K + i) * CHUNK, CHUNK)],
                sc[bk],
            )

    fire_gathers(0, 0)

    def body(g, carry):
        drain(0, sg)

        @pl.when(g > 0)
        def _():
            drain(1, sc)

        fire_gathers(g + 1, 1)
        fire_copyouts(g, 0)
        drain(1, sg)
        drain(0, sc)

        @pl.when(g + 2 < NG)
        def _():
            fire_gathers(g + 2, 0)

        fire_copyouts(g + 1, 1)
        return carry

    lax.fori_loop(0, NG // 2, lambda t, c: body(t * 2, c), 0)
    drain(1, sc)


N_PAIR = N_ROWS // 2
BLK2 = 2048
N_BLK = N_PAIR // BLK2


def _proj_body(e_ref, bd_ref, b2_ref, out_ref):
    out_ref[...] = (
        jnp.dot(e_ref[...], bd_ref[...], preferred_element_type=jnp.float32)
        + b2_ref[...]
    )


def _project(emb2, bd, b2):
    return pl.pallas_call(
        _proj_body,
        grid=(N_BLK,),
        in_specs=[
            pl.BlockSpec((BLK2, 2 * D), lambda i: (i, 0)),
            pl.BlockSpec((2 * D, 2 * D), lambda i: (0, 0)),
            pl.BlockSpec((1, 2 * D), lambda i: (0, 0)),
        ],
        out_specs=pl.BlockSpec((BLK2, 2 * D), lambda i: (i, 0)),
        out_shape=jax.ShapeDtypeStruct((N_PAIR, 2 * D), jnp.float32),
    )(emb2, bd, b2)


def kernel(x, table, W, b):
    idx3 = x.reshape(NW, N_CHUNKS, CHUNK)
    emb = _sc_gather(table, idx3)
    emb2 = emb.reshape(-1).reshape(N_PAIR, 2 * D)
    wt = W.T
    bd = (
        jnp.zeros((2 * D, 2 * D), jnp.float32)
        .at[:D, :D].set(wt)
        .at[D:, D:].set(wt)
    )
    b2 = jnp.concatenate([b, b]).reshape(1, 2 * D)
    out2 = _project(emb2, bd, b2)
    return out2.reshape(B, L, D)
